# Initial kernel scaffold; baseline (speedup 1.0000x reference)
import jax
import jax.numpy as jnp
from jax.experimental import pallas as pl


def kernel(x, edge_index, batch, conv1_w1, conv1_b1, conv1_w2, conv1_b2, bn1_g, bn1_b, conv2_w1, conv2_b1, conv2_w2, conv2_b2, bn2_g, bn2_b, fc_xd_w, fc_xd_b, fc1_w, fc1_b, fc2_w, fc2_b, out_w, out_b):
    raise NotImplementedError("write your pallas kernel here")


# speedup vs baseline: 4.7870x; 4.7870x over previous
import functools

import jax
import jax.numpy as jnp
from jax import lax
from jax.experimental import pallas as pl
from jax.experimental.pallas import tpu as pltpu
from jax.experimental.pallas import tpu_sc as plsc

_N = 10000
_E = 320000
_D = 128
_H = 96
_G = 256

_NC = 2
_NS = 16
_NW = _NC * _NS
_EPW = _E // _NW
_CH = 80
_NCHUNK = _EPW // _CH
_NPAD = 10240
_RPS = _NPAD // _NS
_ZR = 128
_W = 128


def _sc_scatter_add(table, src, dst):
    mesh = plsc.VectorSubcoreMesh(core_axis_name="c", subcore_axis_name="s")

    @functools.partial(
        pl.kernel,
        out_type=jax.ShapeDtypeStruct((_NC * _NPAD, _W), jnp.float32),
        mesh=mesh,
        scratch_types=[
            pltpu.VMEM((_CH,), jnp.int32),
            pltpu.VMEM((_CH,), jnp.int32),
            pltpu.VMEM((_CH, _W), jnp.float32),
            pltpu.VMEM((_ZR, _W), jnp.float32),
            pltpu.VMEM_SHARED((_NPAD, _W), jnp.float32),
            pltpu.SemaphoreType.DMA,
        ],
    )
    def k(t_hbm, src_hbm, dst_hbm, out_hbm, sidx, didx, rows, zbuf, acc, sem):
        cid = lax.axis_index("c")
        sid = lax.axis_index("s")
        wid = sid * _NC + cid

        @pl.loop(0, _ZR)
        def _(i):
            @pl.loop(0, _W, step=16)
            def _(j):
                zbuf[i, pl.ds(j, 16)] = jnp.zeros((16,), jnp.float32)

        @pl.loop(0, _RPS, step=_ZR)
        def _(r):
            pltpu.sync_copy(zbuf, acc.at[pl.ds(sid * _RPS + r, _ZR)])

        plsc.subcore_barrier()

        base = wid * _EPW

        @pl.loop(0, _NCHUNK)
        def _(i):
            off = base + i * _CH
            pltpu.sync_copy(src_hbm.at[pl.ds(off, _CH)], sidx)
            pltpu.sync_copy(dst_hbm.at[pl.ds(off, _CH)], didx)
            pltpu.async_copy(t_hbm.at[sidx], rows, sem).wait()
            pltpu.sync_copy(rows, acc.at[didx], add=True)

        plsc.subcore_barrier()

        pltpu.sync_copy(
            acc.at[pl.ds(sid * _RPS, _RPS)],
            out_hbm.at[pl.ds(cid * _NPAD + sid * _RPS, _RPS)],
        )

    return k(table, src, dst)


def _mish(x):
    sp = jnp.maximum(x, 0.0) + jnp.log1p(jnp.exp(-jnp.abs(x)))
    return x * jnp.tanh(sp)


def _bn(u, g, b):
    m = jnp.mean(u, axis=0, keepdims=True)
    v = jnp.mean((u - m) * (u - m), axis=0, keepdims=True)
    return g * (u - m) * lax.rsqrt(v + 1e-5) + b


def _tc_conv_stage(x, spart, w1, b1, w2, b2, g, bb):
    def body(x_ref, s_ref, w1_ref, b1_ref, w2_ref, b2_ref, g_ref, bb_ref,
             o_ref):
        v = x_ref[...] + s_ref[pl.ds(0, _N), :] + s_ref[pl.ds(_NPAD, _N), :]
        t = jnp.dot(v, w1_ref[...], preferred_element_type=jnp.float32)
        u = jnp.dot(_mish(t + b1_ref[...]), w2_ref[...],
                    preferred_element_type=jnp.float32) + b2_ref[...]
        h = _bn(_mish(u), g_ref[...], bb_ref[...])
        o_ref[...] = jnp.concatenate(
            [h, jnp.zeros((_N, _W - _H), jnp.float32)], axis=1)

    return pl.pallas_call(
        body,
        out_shape=jax.ShapeDtypeStruct((_N, _W), jnp.float32),
    )(x, spart, w1, b1, w2, b2, g, bb)


def _tc_final_stage(h, spart, w1, b1, w2, b2, g, bb, batch,
                    fxw, fxb, f1w, f1b, f2w, f2b, ow, ob):
    def body(h_ref, s_ref, w1_ref, b1_ref, w2_ref, b2_ref, g_ref, bb_ref,
             batch_ref, fxw_ref, fxb_ref, f1w_ref, f1b_ref, f2w_ref, f2b_ref,
             ow_ref, ob_ref, o_ref):
        v = h_ref[...] + s_ref[pl.ds(0, _N), :] + s_ref[pl.ds(_NPAD, _N), :]
        t = jnp.dot(v, w1_ref[...], preferred_element_type=jnp.float32)
        u = jnp.dot(_mish(t + b1_ref[...]), w2_ref[...],
                    preferred_element_type=jnp.float32) + b2_ref[...]
        h2 = _bn(_mish(u), g_ref[...], bb_ref[...])

        gid = lax.broadcasted_iota(jnp.int32, (_G, _N), 0)
        onehot = jnp.where(gid == batch_ref[...], 1.0, 0.0)
        pooled = jnp.dot(onehot, h2, preferred_element_type=jnp.float32,
                         precision=lax.Precision.HIGHEST)

        z = _mish(jnp.dot(pooled, fxw_ref[...],
                          preferred_element_type=jnp.float32) + fxb_ref[...])
        z = _mish(jnp.dot(z, f1w_ref[...],
                          preferred_element_type=jnp.float32) + f1b_ref[...])
        z = _mish(jnp.dot(z, f2w_ref[...],
                          preferred_element_type=jnp.float32) + f2b_ref[...])
        o_ref[...] = jnp.dot(z, ow_ref[...],
                             preferred_element_type=jnp.float32) + ob_ref[...]

    return pl.pallas_call(
        body,
        out_shape=jax.ShapeDtypeStruct((_G, 1), jnp.float32),
    )(h, spart, w1, b1, w2, b2, g, bb, batch,
      fxw, fxb, f1w, f1b, f2w, f2b, ow, ob)


def kernel(x, edge_index, batch,
           conv1_w1, conv1_b1, conv1_w2, conv1_b2, bn1_g, bn1_b,
           conv2_w1, conv2_b1, conv2_w2, conv2_b2, bn2_g, bn2_b,
           fc_xd_w, fc_xd_b, fc1_w, fc1_b, fc2_w, fc2_b, out_w, out_b):
    src = edge_index[0]
    dst = edge_index[1]
    batch_row = batch.reshape(1, _N)
    r = lambda v: v.reshape(1, -1)
    w21 = jnp.pad(conv2_w1, ((0, _W - _H), (0, 0)))

    s1 = _sc_scatter_add(x, src, dst)
    hpad = _tc_conv_stage(x, s1, conv1_w1, r(conv1_b1), conv1_w2,
                          r(conv1_b2), r(bn1_g), r(bn1_b))

    s2 = _sc_scatter_add(hpad, src, dst)
    return _tc_final_stage(hpad, s2, w21, r(conv2_b1), conv2_w2, r(conv2_b2),
                           r(bn2_g), r(bn2_b), batch_row,
                           fc_xd_w, r(fc_xd_b), fc1_w, r(fc1_b),
                           fc2_w, r(fc2_b), out_w, r(out_b))


# speedup vs baseline: 10.0126x; 2.0916x over previous
import functools

import jax
import jax.numpy as jnp
from jax import lax
from jax.experimental import pallas as pl
from jax.experimental.pallas import tpu as pltpu
from jax.experimental.pallas import tpu_sc as plsc

_N = 10000
_E = 320000
_D = 128
_H = 96
_G = 256

_NC = 2
_NS = 16
_NW = _NC * _NS
_EPW = _E // _NW
_CH = 128
_NCHUNK = 80
_EPAD = _NW * _NCHUNK * _CH
_NPAD = 10240
_RPS = _NPAD // _NS
_W = 128


def _sc_scatter_add(table, src, dst):
    mesh = plsc.VectorSubcoreMesh(core_axis_name="c", subcore_axis_name="s")

    @functools.partial(
        pl.kernel,
        out_type=jax.ShapeDtypeStruct((_NC * _NPAD, _W), jnp.float32),
        mesh=mesh,
        scratch_types=[
            pltpu.VMEM((_CH,), jnp.int32),
            pltpu.VMEM((_CH,), jnp.int32),
            pltpu.VMEM((_CH,), jnp.int32),
            pltpu.VMEM((_CH,), jnp.int32),
            pltpu.VMEM((_CH, _W), jnp.float32),
            pltpu.VMEM((_CH, _W), jnp.float32),
            pltpu.VMEM_SHARED((_NPAD, _W), jnp.float32),
            pltpu.SemaphoreType.DMA,
            pltpu.SemaphoreType.DMA,
            pltpu.SemaphoreType.DMA,
            pltpu.SemaphoreType.DMA,
        ],
    )
    def k(t_hbm, src_hbm, dst_hbm, out_hbm, sidx0, didx0, sidx1, didx1,
          rows0, rows1, acc, semi0, semi1, semg0, semg1):
        cid = lax.axis_index("c")
        sid = lax.axis_index("s")
        wid = sid * _NC + cid
        base = wid * _NCHUNK * _CH

        def istart(c, sbuf, dbuf, sem):
            pltpu.async_copy(src_hbm.at[pl.ds(base + c * _CH, _CH)], sbuf, sem)
            pltpu.async_copy(dst_hbm.at[pl.ds(base + c * _CH, _CH)], dbuf, sem)

        def iwait(sbuf, dbuf, sem):
            pltpu.make_async_copy(src_hbm.at[pl.ds(0, _CH)], sbuf, sem).wait()
            pltpu.make_async_copy(dst_hbm.at[pl.ds(0, _CH)], dbuf, sem).wait()

        def gstart(sbuf, buf, sem):
            pltpu.async_copy(t_hbm.at[sbuf], buf, sem)

        def gwait(buf, sem):
            pltpu.make_async_copy(t_hbm.at[sidx0], buf, sem).wait()

        def sadd(buf, dbuf):
            pltpu.sync_copy(buf, acc.at[dbuf], add=True)

        istart(0, sidx0, didx0, semi0)
        istart(1, sidx1, didx1, semi1)

        @pl.loop(0, _CH)
        def _(i):
            @pl.loop(0, _W, step=16)
            def _(j):
                rows1[i, pl.ds(j, 16)] = jnp.zeros((16,), jnp.float32)

        @pl.loop(0, _RPS, step=_CH)
        def _(r):
            pltpu.sync_copy(rows1, acc.at[pl.ds(sid * _RPS + r, _CH)])

        iwait(sidx0, didx0, semi0)
        plsc.subcore_barrier()

        gstart(sidx0, rows0, semg0)

        @pl.loop(0, _NCHUNK - 2, step=2)
        def _(g):
            iwait(sidx1, didx1, semi1)
            gwait(rows0, semg0)
            gstart(sidx1, rows1, semg1)
            sadd(rows0, didx0)
            istart(g + 2, sidx0, didx0, semi0)
            gwait(rows1, semg1)
            iwait(sidx0, didx0, semi0)
            gstart(sidx0, rows0, semg0)
            sadd(rows1, didx1)
            istart(g + 3, sidx1, didx1, semi1)

        iwait(sidx1, didx1, semi1)
        gwait(rows0, semg0)
        gstart(sidx1, rows1, semg1)
        sadd(rows0, didx0)
        gwait(rows1, semg1)
        sadd(rows1, didx1)

        plsc.subcore_barrier()

        pltpu.sync_copy(
            acc.at[pl.ds(sid * _RPS, _RPS)],
            out_hbm.at[pl.ds(cid * _NPAD + sid * _RPS, _RPS)],
        )

    return k(table, src, dst)


def _mish(x):
    sp = jnp.maximum(x, 0.0) + jnp.log1p(jnp.exp(-jnp.abs(x)))
    return x * jnp.tanh(sp)


def _bn(u, g, b):
    m = jnp.mean(u, axis=0, keepdims=True)
    v = jnp.mean((u - m) * (u - m), axis=0, keepdims=True)
    return g * (u - m) * lax.rsqrt(v + 1e-5) + b


def _tc_conv_stage(x, spart, w1, b1, w2, b2, g, bb):
    def body(x_ref, s_ref, w1_ref, b1_ref, w2_ref, b2_ref, g_ref, bb_ref,
             o_ref):
        v = x_ref[...] + s_ref[pl.ds(0, _N), :] + s_ref[pl.ds(_NPAD, _N), :]
        t = jnp.dot(v, w1_ref[...], preferred_element_type=jnp.float32)
        u = jnp.dot(_mish(t + b1_ref[...]), w2_ref[...],
                    preferred_element_type=jnp.float32) + b2_ref[...]
        h = _bn(_mish(u), g_ref[...], bb_ref[...])
        o_ref[...] = jnp.concatenate(
            [h, jnp.zeros((_N, _W - _H), jnp.float32)], axis=1)

    return pl.pallas_call(
        body,
        out_shape=jax.ShapeDtypeStruct((_N, _W), jnp.float32),
    )(x, spart, w1, b1, w2, b2, g, bb)


def _tc_final_stage(h, spart, w1, b1, w2, b2, g, bb, batch,
                    fxw, fxb, f1w, f1b, f2w, f2b, ow, ob):
    def body(h_ref, s_ref, w1_ref, b1_ref, w2_ref, b2_ref, g_ref, bb_ref,
             batch_ref, fxw_ref, fxb_ref, f1w_ref, f1b_ref, f2w_ref, f2b_ref,
             ow_ref, ob_ref, o_ref):
        v = h_ref[...] + s_ref[pl.ds(0, _N), :] + s_ref[pl.ds(_NPAD, _N), :]
        t = jnp.dot(v, w1_ref[...], preferred_element_type=jnp.float32)
        u = jnp.dot(_mish(t + b1_ref[...]), w2_ref[...],
                    preferred_element_type=jnp.float32) + b2_ref[...]
        h2 = _bn(_mish(u), g_ref[...], bb_ref[...])

        gid = lax.broadcasted_iota(jnp.int32, (_G, _N), 0)
        onehot = jnp.where(gid == batch_ref[...], 1.0, 0.0)
        pooled = jnp.dot(onehot, h2, preferred_element_type=jnp.float32,
                         precision=lax.Precision.HIGHEST)

        z = _mish(jnp.dot(pooled, fxw_ref[...],
                          preferred_element_type=jnp.float32) + fxb_ref[...])
        z = _mish(jnp.dot(z, f1w_ref[...],
                          preferred_element_type=jnp.float32) + f1b_ref[...])
        z = _mish(jnp.dot(z, f2w_ref[...],
                          preferred_element_type=jnp.float32) + f2b_ref[...])
        o_ref[...] = jnp.dot(z, ow_ref[...],
                             preferred_element_type=jnp.float32) + ob_ref[...]

    return pl.pallas_call(
        body,
        out_shape=jax.ShapeDtypeStruct((_G, 1), jnp.float32),
    )(h, spart, w1, b1, w2, b2, g, bb, batch,
      fxw, fxb, f1w, f1b, f2w, f2b, ow, ob)


def kernel(x, edge_index, batch,
           conv1_w1, conv1_b1, conv1_w2, conv1_b2, bn1_g, bn1_b,
           conv2_w1, conv2_b1, conv2_w2, conv2_b2, bn2_g, bn2_b,
           fc_xd_w, fc_xd_b, fc1_w, fc1_b, fc2_w, fc2_b, out_w, out_b):
    ar = jnp.arange(_EPAD - _E, dtype=jnp.int32)
    src = jnp.concatenate([edge_index[0], ar % _N])
    dst = jnp.concatenate([edge_index[1], _N + ar % (_NPAD - _N)])
    batch_row = batch.reshape(1, _N)
    r = lambda v: v.reshape(1, -1)
    w21 = jnp.pad(conv2_w1, ((0, _W - _H), (0, 0)))

    s1 = _sc_scatter_add(x, src, dst)
    hpad = _tc_conv_stage(x, s1, conv1_w1, r(conv1_b1), conv1_w2,
                          r(conv1_b2), r(bn1_g), r(bn1_b))

    s2 = _sc_scatter_add(hpad, src, dst)
    return _tc_final_stage(hpad, s2, w21, r(conv2_b1), conv2_w2, r(conv2_b2),
                           r(bn2_g), r(bn2_b), batch_row,
                           fc_xd_w, r(fc_xd_b), fc1_w, r(fc1_b),
                           fc2_w, r(fc2_b), out_w, r(out_b))


# speedup vs baseline: 10.1089x; 1.0096x over previous
import functools

import jax
import jax.numpy as jnp
from jax import lax
from jax.experimental import pallas as pl
from jax.experimental.pallas import tpu as pltpu
from jax.experimental.pallas import tpu_sc as plsc

_N = 10000
_E = 320000
_D = 128
_H = 96
_G = 256

_NC = 2
_NS = 16
_NW = _NC * _NS
_EPW = _E // _NW
_CH = 128
_NCHUNK = 80
_EPAD = _NW * _NCHUNK * _CH
_NPAD = 10240
_RPS = _NPAD // _NS
_W = 128


def _sc_scatter_add(table, src, dst):
    mesh = plsc.VectorSubcoreMesh(core_axis_name="c", subcore_axis_name="s")

    @functools.partial(
        pl.kernel,
        out_type=jax.ShapeDtypeStruct((_NC * _NPAD, _W), jnp.float32),
        mesh=mesh,
        scratch_types=[
            pltpu.VMEM((_CH,), jnp.int32),
            pltpu.VMEM((_CH,), jnp.int32),
            pltpu.VMEM((_CH,), jnp.int32),
            pltpu.VMEM((_CH,), jnp.int32),
            pltpu.VMEM((_CH, _W), jnp.float32),
            pltpu.VMEM((_CH, _W), jnp.float32),
            pltpu.VMEM_SHARED((_NPAD, _W), jnp.float32),
            pltpu.SemaphoreType.DMA,
            pltpu.SemaphoreType.DMA,
            pltpu.SemaphoreType.DMA,
            pltpu.SemaphoreType.DMA,
        ],
    )
    def k(t_hbm, src_hbm, dst_hbm, out_hbm, sidx0, didx0, sidx1, didx1,
          rows0, rows1, acc, semi0, semi1, semg0, semg1):
        cid = lax.axis_index("c")
        sid = lax.axis_index("s")
        wid = sid * _NC + cid
        base = wid * _NCHUNK * _CH

        def istart(c, sbuf, dbuf, sem):
            pltpu.async_copy(src_hbm.at[pl.ds(base + c * _CH, _CH)], sbuf, sem)
            pltpu.async_copy(dst_hbm.at[pl.ds(base + c * _CH, _CH)], dbuf, sem)

        def iwait(sbuf, dbuf, sem):
            pltpu.make_async_copy(src_hbm.at[pl.ds(0, _CH)], sbuf, sem).wait()
            pltpu.make_async_copy(dst_hbm.at[pl.ds(0, _CH)], dbuf, sem).wait()

        def gstart(sbuf, buf, sem):
            pltpu.async_copy(t_hbm.at[sbuf], buf, sem)

        def gwait(buf, sem):
            pltpu.make_async_copy(t_hbm.at[sidx0], buf, sem).wait()

        def sadd(buf, dbuf):
            pltpu.sync_copy(buf, acc.at[dbuf], add=True)

        istart(0, sidx0, didx0, semi0)
        istart(1, sidx1, didx1, semi1)

        @pl.loop(0, _CH)
        def _(i):
            @pl.loop(0, _W, step=16)
            def _(j):
                rows1[i, pl.ds(j, 16)] = jnp.zeros((16,), jnp.float32)

        @pl.loop(0, _RPS, step=_CH)
        def _(r):
            pltpu.sync_copy(rows1, acc.at[pl.ds(sid * _RPS + r, _CH)])

        iwait(sidx0, didx0, semi0)
        plsc.subcore_barrier()

        gstart(sidx0, rows0, semg0)

        @pl.loop(0, _NCHUNK - 2, step=2)
        def _(g):
            iwait(sidx1, didx1, semi1)
            gwait(rows0, semg0)
            gstart(sidx1, rows1, semg1)
            sadd(rows0, didx0)
            istart(g + 2, sidx0, didx0, semi0)
            gwait(rows1, semg1)
            iwait(sidx0, didx0, semi0)
            gstart(sidx0, rows0, semg0)
            sadd(rows1, didx1)
            istart(g + 3, sidx1, didx1, semi1)

        iwait(sidx1, didx1, semi1)
        gwait(rows0, semg0)
        gstart(sidx1, rows1, semg1)
        sadd(rows0, didx0)
        gwait(rows1, semg1)
        sadd(rows1, didx1)

        plsc.subcore_barrier()

        pltpu.sync_copy(
            acc.at[pl.ds(sid * _RPS, _RPS)],
            out_hbm.at[pl.ds(cid * _NPAD + sid * _RPS, _RPS)],
        )

    return k(table, src, dst)


def _mish(x):
    u = jnp.exp(jnp.minimum(x, 20.0))
    w = u * u + 2.0 * u
    return x * jnp.where(x > 20.0, 1.0, w / (w + 2.0))


def _bn(u, g, b):
    m = jnp.mean(u, axis=0, keepdims=True)
    v = jnp.mean((u - m) * (u - m), axis=0, keepdims=True)
    return g * (u - m) * lax.rsqrt(v + 1e-5) + b


def _tc_conv_stage(x, spart, w1, b1, w2, b2, g, bb):
    def body(x_ref, s_ref, w1_ref, b1_ref, w2_ref, b2_ref, g_ref, bb_ref,
             o_ref):
        v = x_ref[...] + s_ref[pl.ds(0, _N), :] + s_ref[pl.ds(_NPAD, _N), :]
        t = jnp.dot(v, w1_ref[...], preferred_element_type=jnp.float32)
        u = jnp.dot(_mish(t + b1_ref[...]), w2_ref[...],
                    preferred_element_type=jnp.float32) + b2_ref[...]
        h = _bn(_mish(u), g_ref[...], bb_ref[...])
        o_ref[...] = jnp.concatenate(
            [h, jnp.zeros((_N, _W - _H), jnp.float32)], axis=1)

    return pl.pallas_call(
        body,
        out_shape=jax.ShapeDtypeStruct((_N, _W), jnp.float32),
    )(x, spart, w1, b1, w2, b2, g, bb)


def _tc_final_stage(h, spart, w1, b1, w2, b2, g, bb, batch,
                    fxw, fxb, f1w, f1b, f2w, f2b, ow, ob):
    def body(h_ref, s_ref, w1_ref, b1_ref, w2_ref, b2_ref, g_ref, bb_ref,
             batch_ref, fxw_ref, fxb_ref, f1w_ref, f1b_ref, f2w_ref, f2b_ref,
             ow_ref, ob_ref, o_ref):
        v = h_ref[...] + s_ref[pl.ds(0, _N), :] + s_ref[pl.ds(_NPAD, _N), :]
        t = jnp.dot(v, w1_ref[...], preferred_element_type=jnp.float32)
        u = jnp.dot(_mish(t + b1_ref[...]), w2_ref[...],
                    preferred_element_type=jnp.float32) + b2_ref[...]
        h2 = _bn(_mish(u), g_ref[...], bb_ref[...])

        gid = lax.broadcasted_iota(jnp.int32, (_G, _N), 0)
        onehot = jnp.where(gid == batch_ref[...], 1.0, 0.0)
        pooled = jnp.dot(onehot, h2, preferred_element_type=jnp.float32,
                         precision=lax.Precision.HIGHEST)

        z = _mish(jnp.dot(pooled, fxw_ref[...],
                          preferred_element_type=jnp.float32) + fxb_ref[...])
        z = _mish(jnp.dot(z, f1w_ref[...],
                          preferred_element_type=jnp.float32) + f1b_ref[...])
        z = _mish(jnp.dot(z, f2w_ref[...],
                          preferred_element_type=jnp.float32) + f2b_ref[...])
        o_ref[...] = jnp.dot(z, ow_ref[...],
                             preferred_element_type=jnp.float32) + ob_ref[...]

    return pl.pallas_call(
        body,
        out_shape=jax.ShapeDtypeStruct((_G, 1), jnp.float32),
    )(h, spart, w1, b1, w2, b2, g, bb, batch,
      fxw, fxb, f1w, f1b, f2w, f2b, ow, ob)


def kernel(x, edge_index, batch,
           conv1_w1, conv1_b1, conv1_w2, conv1_b2, bn1_g, bn1_b,
           conv2_w1, conv2_b1, conv2_w2, conv2_b2, bn2_g, bn2_b,
           fc_xd_w, fc_xd_b, fc1_w, fc1_b, fc2_w, fc2_b, out_w, out_b):
    ar = jnp.arange(_EPAD - _E, dtype=jnp.int32)
    src = jnp.concatenate([edge_index[0], ar % _N])
    dst = jnp.concatenate([edge_index[1], _N + ar % (_NPAD - _N)])
    batch_row = batch.reshape(1, _N)
    r = lambda v: v.reshape(1, -1)
    w21 = jnp.pad(conv2_w1, ((0, _W - _H), (0, 0)))

    s1 = _sc_scatter_add(x, src, dst)
    hpad = _tc_conv_stage(x, s1, conv1_w1, r(conv1_b1), conv1_w2,
                          r(conv1_b2), r(bn1_g), r(bn1_b))

    s2 = _sc_scatter_add(hpad, src, dst)
    return _tc_final_stage(hpad, s2, w21, r(conv2_b1), conv2_w2, r(conv2_b2),
                           r(bn2_g), r(bn2_b), batch_row,
                           fc_xd_w, r(fc_xd_b), fc1_w, r(fc1_b),
                           fc2_w, r(fc2_b), out_w, r(out_b))
